# baseline (speedup 1.0000x reference)
import functools

import jax
import jax.numpy as jnp
from jax import lax
from jax.experimental import pallas as pl
from jax.experimental.pallas import tpu as pltpu
from jax.experimental.pallas import tpu_sc as plsc

CH = 128
K = 8
SUP = CH * K
BLK = 4096
HALF = 512000


def _tc_project_table(tableT, V, b2d):
    d, vocab = tableT.shape
    d_out = V.shape[1]

    def body(tlo_ref, thi_ref, v_ref, b_ref, o_ref):
        def proj(t):
            return lax.dot_general(
                t, v_ref[...],
                dimension_numbers=(((0,), (0,)), ((), ())),
                preferred_element_type=jnp.float32,
            ) + b_ref[...]
        o_ref[...] = jnp.concatenate(
            [proj(tlo_ref[...]), proj(thi_ref[...])], axis=1)

    nblk = HALF // BLK
    last = (vocab - 1) // BLK
    return pl.pallas_call(
        body,
        grid=(nblk,),
        in_specs=[
            pl.BlockSpec((d, BLK), lambda i: (0, i)),
            pl.BlockSpec((d, BLK), lambda i: (0, jnp.minimum(i + nblk, last))),
            pl.BlockSpec((d, d_out), lambda i: (0, 0)),
            pl.BlockSpec((1, d_out), lambda i: (0, 0)),
        ],
        out_specs=pl.BlockSpec((BLK, 2 * d_out), lambda i: (i, 0)),
        out_shape=jax.ShapeDtypeStruct((HALF, 2 * d_out), jnp.float32),
    )(tableT, tableT, V, b2d)


def _sc_gather(p_rows, idx2d, n_rows):
    info = plsc.get_sparse_core_info()
    nc, ns = info.num_cores, info.num_subcores
    nw = nc * ns
    d = p_rows.shape[1]
    rows_per_w = n_rows // nw
    n_sup = rows_per_w // SUP

    mesh = plsc.VectorSubcoreMesh(core_axis_name="c", subcore_axis_name="s")

    @functools.partial(
        pl.kernel,
        out_type=jax.ShapeDtypeStruct((n_rows, d), jnp.float32),
        mesh=mesh,
        scratch_types=[
            pltpu.VMEM((K, CH), jnp.int32),
            pltpu.VMEM((SUP, d), jnp.float32),
            pltpu.SemaphoreType.DMA,
        ],
        compiler_params=pltpu.CompilerParams(use_tc_tiling_on_sc=False),
    )
    def gather_kernel(table_hbm, idx_hbm, out_hbm, idx_v, rows_v, gsem):
        wid = lax.axis_index("s") * nc + lax.axis_index("c")
        row0 = wid * rows_per_w
        irow0 = wid * (rows_per_w // CH)

        def super_iter(ob, _):
            pltpu.sync_copy(idx_hbm.at[pl.ds(irow0 + ob * K, K)], idx_v)
            copies = []
            for j in range(K):
                copies.append(pltpu.async_copy(
                    table_hbm.at[idx_v.at[j]],
                    rows_v.at[pl.ds(j * CH, CH)],
                    gsem,
                ))
            for c in copies:
                c.wait()
            pltpu.sync_copy(rows_v, out_hbm.at[pl.ds(row0 + ob * SUP, SUP)])
            return _

        lax.fori_loop(0, n_sup, super_iter, 0)

    return gather_kernel(p_rows, idx2d)


def _tc_untranspose(emb3, batch, seq, d):
    bb = 1024

    def body(e_ref, o_ref):
        e = e_ref[0]
        o_ref[0] = e[:, :d].T
        o_ref[1] = e[:, d:].T

    return pl.pallas_call(
        body,
        grid=(seq // 2, batch // bb),
        in_specs=[pl.BlockSpec((1, bb, 2 * d), lambda j, i: (j, i, 0))],
        out_specs=pl.BlockSpec((2, d, bb), lambda j, i: (j, 0, i)),
        out_shape=jax.ShapeDtypeStruct((seq, d, batch), jnp.float32),
    )(emb3)


def kernel(x, table, W, b):
    batch, seq = x.shape
    n = batch * seq
    vocab, d = table.shape
    xp = x.reshape(batch, seq // 2, 2).transpose(1, 0, 2).reshape(n)
    idx2d = jnp.where(xp < HALF, 2 * xp, 2 * (xp - HALF) + 1).reshape(
        n // CH, CH)
    p2 = _tc_project_table(table.T, W.T, b.reshape(1, -1))
    p_rows = p2.reshape(2 * HALF, d)
    emb = _sc_gather(p_rows, idx2d, n)
    emb3 = emb.reshape(seq // 2, batch, 2 * d)
    out_t = _tc_untranspose(emb3, batch, seq, d)
    return out_t.transpose(2, 0, 1)


# speedup vs baseline: 1.1975x; 1.1975x over previous
import functools

import jax
import jax.numpy as jnp
from jax import lax
from jax.experimental import pallas as pl
from jax.experimental.pallas import tpu as pltpu
from jax.experimental.pallas import tpu_sc as plsc

CH = 128
K = 5
SUP = CH * K
BLK = 4096
HALF = 512000


def _tc_project_table(tableT, V, b2d):
    d, vocab = tableT.shape
    d_out = V.shape[1]

    def body(tlo_ref, thi_ref, v_ref, b_ref, o_ref):
        def proj(t):
            return lax.dot_general(
                t, v_ref[...],
                dimension_numbers=(((0,), (0,)), ((), ())),
                preferred_element_type=jnp.float32,
            ) + b_ref[...]
        o_ref[...] = jnp.concatenate(
            [proj(tlo_ref[...]), proj(thi_ref[...])], axis=1)

    nblk = HALF // BLK
    last = (vocab - 1) // BLK
    return pl.pallas_call(
        body,
        grid=(nblk,),
        in_specs=[
            pl.BlockSpec((d, BLK), lambda i: (0, i)),
            pl.BlockSpec((d, BLK), lambda i: (0, jnp.minimum(i + nblk, last))),
            pl.BlockSpec((d, d_out), lambda i: (0, 0)),
            pl.BlockSpec((1, d_out), lambda i: (0, 0)),
        ],
        out_specs=pl.BlockSpec((BLK, 2 * d_out), lambda i: (i, 0)),
        out_shape=jax.ShapeDtypeStruct((HALF, 2 * d_out), jnp.float32),
    )(tableT, tableT, V, b2d)


def _sc_gather(p_rows, idx2d, n_rows):
    info = plsc.get_sparse_core_info()
    nc, ns = info.num_cores, info.num_subcores
    nw = nc * ns
    d = p_rows.shape[1]
    rows_per_w = n_rows // nw
    n_sup = rows_per_w // SUP

    mesh = plsc.VectorSubcoreMesh(core_axis_name="c", subcore_axis_name="s")

    @functools.partial(
        pl.kernel,
        out_type=jax.ShapeDtypeStruct((n_rows, d), jnp.float32),
        mesh=mesh,
        scratch_types=[
            pltpu.VMEM((2 * K, CH), jnp.int32),
            pltpu.VMEM((2 * SUP, d), jnp.float32),
            pltpu.SemaphoreType.DMA,
            pltpu.SemaphoreType.DMA,
            pltpu.SemaphoreType.DMA,
        ],
        compiler_params=pltpu.CompilerParams(use_tc_tiling_on_sc=False),
    )
    def gather_kernel(table_hbm, idx_hbm, out_hbm, idx_v, rows_v,
                      gsem, isem, ssem):
        wid = lax.axis_index("s") * nc + lax.axis_index("c")
        row0 = wid * rows_per_w
        irow0 = wid * (rows_per_w // CH)

        pltpu.sync_copy(idx_hbm.at[pl.ds(irow0, K)], idx_v.at[pl.ds(0, K)])
        pltpu.async_copy(idx_hbm.at[pl.ds(irow0 + K, K)],
                         idx_v.at[pl.ds(K, K)], isem)

        def super_iter(ob, _):
            ioff = (ob % 2) * K
            roff = (ob % 2) * SUP

            @pl.when(ob >= 1)
            def _idx_ready():
                pltpu.make_async_copy(
                    idx_hbm.at[pl.ds(irow0, K)],
                    idx_v.at[pl.ds(ioff, K)], isem).wait()

            @pl.when(ob >= 2)
            def _buf_free():
                pltpu.make_async_copy(
                    rows_v.at[pl.ds(roff, SUP)],
                    out_hbm.at[pl.ds(row0, SUP)], ssem).wait()

            copies = []
            for j in range(K):
                copies.append(pltpu.async_copy(
                    table_hbm.at[idx_v.at[ioff + j]],
                    rows_v.at[pl.ds(roff + j * CH, CH)],
                    gsem,
                ))
            for c in copies:
                c.wait()

            @pl.when(ob + 2 < n_sup)
            def _idx_prefetch():
                pltpu.async_copy(idx_hbm.at[pl.ds(irow0 + (ob + 2) * K, K)],
                                 idx_v.at[pl.ds(ioff, K)], isem)

            pltpu.async_copy(rows_v.at[pl.ds(roff, SUP)],
                             out_hbm.at[pl.ds(row0 + ob * SUP, SUP)], ssem)
            return _

        lax.fori_loop(0, n_sup, super_iter, 0)
        for _ in range(2):
            pltpu.make_async_copy(rows_v.at[pl.ds(0, SUP)],
                                  out_hbm.at[pl.ds(row0, SUP)], ssem).wait()

    return gather_kernel(p_rows, idx2d)


def _tc_untranspose(emb3, batch, seq, d):
    bb = 4096

    def body(e_ref, o_ref):
        e = e_ref[0]
        o_ref[0] = e[:, :d].T
        o_ref[1] = e[:, d:].T

    return pl.pallas_call(
        body,
        grid=(seq // 2, batch // bb),
        in_specs=[pl.BlockSpec((1, bb, 2 * d), lambda j, i: (j, i, 0))],
        out_specs=pl.BlockSpec((2, d, bb), lambda j, i: (j, 0, i)),
        out_shape=jax.ShapeDtypeStruct((seq, d, batch), jnp.float32),
    )(emb3)


def kernel(x, table, W, b):
    batch, seq = x.shape
    n = batch * seq
    vocab, d = table.shape
    xp = x.reshape(batch, seq // 2, 2).transpose(1, 0, 2).reshape(n)
    idx2d = jnp.where(xp < HALF, 2 * xp, 2 * (xp - HALF) + 1).reshape(
        n // CH, CH)
    p2 = _tc_project_table(table.T, W.T, b.reshape(1, -1))
    p_rows = p2.reshape(2 * HALF, d)
    emb = _sc_gather(p_rows, idx2d, n)
    emb3 = emb.reshape(seq // 2, batch, 2 * d)
    out_t = _tc_untranspose(emb3, batch, seq, d)
    return out_t.transpose(2, 0, 1)
